# baseline (speedup 1.0000x reference)
import functools

import jax
import jax.numpy as jnp
from jax.experimental import pallas as pl
from jax.experimental.pallas import tpu as pltpu
from jax.experimental.pallas import tpu_sc as plsc

HIDDEN = 1024
N_HEADS = 16
N_KV = 4
HEAD_DIM = 64
FFN = 1024
E = 8
TOP_K = 2
SEQ = 2048
EPS = 1e-5
THETA = 10000.0

BQ = 512
BKV = 512
BS = 512
MOE_T = 256
N_ASSIGN = SEQ * TOP_K
NT = N_ASSIGN // MOE_T + E
PAD_N = NT * MOE_T


def _bdot(a, b):
    return jax.lax.dot_general(
        a.astype(jnp.bfloat16), b.astype(jnp.bfloat16),
        (((1,), (0,)), ((), ())), preferred_element_type=jnp.float32)


def _bdot_t(a, b):
    return jax.lax.dot_general(
        a.astype(jnp.bfloat16), b.astype(jnp.bfloat16),
        (((1,), (1,)), ((), ())), preferred_element_type=jnp.float32)


def _rms(x):
    return x * jax.lax.rsqrt(jnp.mean(x * x, axis=-1, keepdims=True) + EPS)


def _qkv_kernel(pos_ref, h_ref, ln1_ref, qkvw_ref, q_ref, k_ref, v_ref):
    h = h_ref[...]
    hn = _rms(h) * ln1_ref[0]
    qkv = _bdot_t(hn, qkvw_ref[...])
    half = HEAD_DIM // 2
    inv = THETA ** (-(jax.lax.iota(jnp.int32, half).astype(jnp.float32) * 2.0 / HEAD_DIM))
    pos = pos_ref[0].astype(jnp.float32)
    freqs = pos[:, None] * inv[None, :]
    cos = jnp.cos(freqs)
    sin = jnp.sin(freqs)
    scale = HEAD_DIM ** -0.5
    for hh in range(N_HEADS):
        b = hh * HEAD_DIM
        x1 = qkv[:, b:b + half]
        x2 = qkv[:, b + half:b + HEAD_DIM]
        q_ref[:, b:b + half] = ((x1 * cos - x2 * sin) * scale).astype(jnp.bfloat16)
        q_ref[:, b + half:b + HEAD_DIM] = ((x2 * cos + x1 * sin) * scale).astype(jnp.bfloat16)
    for hh in range(N_KV):
        b = N_HEADS * HEAD_DIM + hh * HEAD_DIM
        x1 = qkv[:, b:b + half]
        x2 = qkv[:, b + half:b + HEAD_DIM]
        k_ref[hh, :half, :] = (x1 * cos - x2 * sin).T.astype(jnp.bfloat16)
        k_ref[hh, half:, :] = (x2 * cos + x1 * sin).T.astype(jnp.bfloat16)
        vb = (N_HEADS + N_KV) * HEAD_DIM + hh * HEAD_DIM
        v_ref[hh, :, :HEAD_DIM] = qkv[:, vb:vb + HEAD_DIM].astype(jnp.bfloat16)
        v_ref[hh, :, HEAD_DIM:] = jnp.concatenate(
            [jnp.ones((BS, 1), jnp.bfloat16),
             jnp.zeros((BS, 127 - HEAD_DIM), jnp.bfloat16)], axis=1)


def _qkv(positions, hidden_states, ln1_w, qkv_w):
    nb = SEQ // BS
    return pl.pallas_call(
        _qkv_kernel,
        grid=(nb,),
        in_specs=[
            pl.BlockSpec((1, BS), lambda i: (0, i)),
            pl.BlockSpec((BS, HIDDEN), lambda i: (i, 0)),
            pl.BlockSpec((1, HIDDEN), lambda i: (0, 0)),
            pl.BlockSpec(((N_HEADS + 2 * N_KV) * HEAD_DIM, HIDDEN), lambda i: (0, 0)),
        ],
        out_specs=(
            pl.BlockSpec((BS, HIDDEN), lambda i: (i, 0)),
            pl.BlockSpec((N_KV, HEAD_DIM, BS), lambda i: (0, 0, i)),
            pl.BlockSpec((N_KV, BS, 128), lambda i: (0, i, 0)),
        ),
        out_shape=(
            jax.ShapeDtypeStruct((SEQ, HIDDEN), jnp.bfloat16),
            jax.ShapeDtypeStruct((N_KV, HEAD_DIM, SEQ), jnp.bfloat16),
            jax.ShapeDtypeStruct((N_KV, SEQ, 128), jnp.bfloat16),
        ),
    )(positions.reshape(1, SEQ), hidden_states, ln1_w.reshape(1, HIDDEN), qkv_w)


def _attn_kernel(q_ref, kT_ref, v_ref, o_ref, acc_ref, m_ref):
    qb = pl.program_id(0)
    j = pl.program_id(1)

    @pl.when(j == 0)
    def _init():
        m_ref[...] = jnp.full_like(m_ref, -1e30)
        acc_ref[...] = jnp.zeros_like(acc_ref)

    def _heads(masked):
        if masked:
            rows = qb * BQ + jax.lax.broadcasted_iota(jnp.int32, (BQ, BKV), 0)
            cols = j * BKV + jax.lax.broadcasted_iota(jnp.int32, (BQ, BKV), 1)
            bias = jnp.where(cols <= rows, jnp.float32(0), jnp.float32(-1e30))
        for hh in range(N_HEADS):
            g = hh // (N_HEADS // N_KV)
            b = hh * HEAD_DIM
            q = q_ref[:, b:b + HEAD_DIM]
            s = _bdot(q, kT_ref[g])
            if masked:
                s = s + bias
            m_prev = m_ref[hh]
            m_cur = jnp.max(s, axis=-1, keepdims=True)
            m_new = jnp.maximum(m_prev, jnp.broadcast_to(m_cur, m_prev.shape))
            alpha = jnp.exp(m_prev[:, :1] - m_new[:, :1])
            p = jnp.exp(s - m_new[:, :1])
            acc_ref[hh] = acc_ref[hh] * alpha + _bdot(p, v_ref[g])
            m_ref[hh] = m_new

    @pl.when(j < qb)
    def _body():
        _heads(False)

    @pl.when(j == qb)
    def _diag():
        _heads(True)

    @pl.when(j == pl.num_programs(1) - 1)
    def _fin():
        for hh in range(N_HEADS):
            b = hh * HEAD_DIM
            a = acc_ref[hh]
            o_ref[:, b:b + HEAD_DIM] = (a[:, :HEAD_DIM] / a[:, HEAD_DIM:HEAD_DIM + 1]
                                        ).astype(jnp.bfloat16)


def _attention(q, kT, v):
    nq = SEQ // BQ
    nkv = SEQ // BKV
    return pl.pallas_call(
        _attn_kernel,
        grid=(nq, nkv),
        in_specs=[
            pl.BlockSpec((BQ, HIDDEN), lambda i, j: (i, 0)),
            pl.BlockSpec((N_KV, HEAD_DIM, BKV), lambda i, j: (0, 0, jnp.minimum(j, i))),
            pl.BlockSpec((N_KV, BKV, 128), lambda i, j: (0, jnp.minimum(j, i), 0)),
        ],
        out_specs=pl.BlockSpec((BQ, HIDDEN), lambda i, j: (i, 0)),
        out_shape=jax.ShapeDtypeStruct((SEQ, HIDDEN), jnp.bfloat16),
        scratch_shapes=[
            pltpu.VMEM((N_HEADS, BQ, 128), jnp.float32),
            pltpu.VMEM((N_HEADS, BQ, 128), jnp.float32),
        ],
    )(q, kT, v)


def _mid_kernel(ao_ref, h0_ref, ow_ref, lnr_ref, w13_ref, w2_ref, lnp_ref,
                gw_ref, rm_ref, hm_ref, probs_ref):
    ao = ao_ref[...]
    ra = h0_ref[...] + _bdot_t(ao, ow_ref[...])
    hr = _rms(ra) * lnr_ref[0]
    g13 = _bdot_t(hr, w13_ref[...])
    g = g13[:, :HIDDEN]
    act = (g / (1.0 + jnp.exp(-g))) * g13[:, HIDDEN:]
    rm_ref[...] = ra + _bdot_t(act, w2_ref[...])
    hm = _rms(ra) * lnp_ref[0]
    hm_ref[...] = hm
    logits = jax.lax.dot_general(hm, gw_ref[...], (((1,), (1,)), ((), ())),
                                 preferred_element_type=jnp.float32)
    mx = jnp.max(logits, axis=-1, keepdims=True)
    ex = jnp.exp(logits - mx)
    probs_ref[...] = ex / jnp.sum(ex, axis=-1, keepdims=True)


def _mid(attn_o, hidden_states, o_w, ln_res_w, res_w13, res_w2, ln_post_w, gate_w):
    nb = SEQ // BS
    return pl.pallas_call(
        _mid_kernel,
        grid=(nb,),
        in_specs=[
            pl.BlockSpec((BS, HIDDEN), lambda i: (i, 0)),
            pl.BlockSpec((BS, HIDDEN), lambda i: (i, 0)),
            pl.BlockSpec((HIDDEN, N_HEADS * HEAD_DIM), lambda i: (0, 0)),
            pl.BlockSpec((1, HIDDEN), lambda i: (0, 0)),
            pl.BlockSpec((2 * HIDDEN, HIDDEN), lambda i: (0, 0)),
            pl.BlockSpec((HIDDEN, HIDDEN), lambda i: (0, 0)),
            pl.BlockSpec((1, HIDDEN), lambda i: (0, 0)),
            pl.BlockSpec((E, HIDDEN), lambda i: (0, 0)),
        ],
        out_specs=(
            pl.BlockSpec((BS, HIDDEN), lambda i: (i, 0)),
            pl.BlockSpec((BS, HIDDEN), lambda i: (i, 0)),
            pl.BlockSpec((BS, E), lambda i: (i, 0)),
        ),
        out_shape=(
            jax.ShapeDtypeStruct((SEQ, HIDDEN), jnp.float32),
            jax.ShapeDtypeStruct((SEQ, HIDDEN), jnp.float32),
            jax.ShapeDtypeStruct((SEQ, E), jnp.float32),
        ),
    )(attn_o, hidden_states, o_w, ln_res_w.reshape(1, HIDDEN), res_w13,
      res_w2, ln_post_w.reshape(1, HIDDEN), gate_w)


def _moe_kernel(te_ref, x_ref, ws_ref, w2s_ref, y_ref):
    x = x_ref[...]
    g13 = _bdot_t(x, ws_ref[0])
    g = g13[:, :FFN]
    act = (g / (1.0 + jnp.exp(-g))) * g13[:, FFN:]
    y_ref[...] = _bdot_t(act, w2s_ref[0])


def _moe_grouped(x_sorted, tile_expert, ws, w2s):
    grid_spec = pltpu.PrefetchScalarGridSpec(
        num_scalar_prefetch=1,
        grid=(NT,),
        in_specs=[
            pl.BlockSpec((MOE_T, HIDDEN), lambda t, te: (t, 0)),
            pl.BlockSpec((1, 2 * FFN, HIDDEN), lambda t, te: (te[t], 0, 0)),
            pl.BlockSpec((1, HIDDEN, FFN), lambda t, te: (te[t], 0, 0)),
        ],
        out_specs=pl.BlockSpec((MOE_T, HIDDEN), lambda t, te: (t, 0)),
    )
    return pl.pallas_call(
        _moe_kernel,
        grid_spec=grid_spec,
        out_shape=jax.ShapeDtypeStruct((PAD_N, HIDDEN), jnp.float32),
    )(tile_expert, x_sorted, ws, w2s)


def _sc_gather(table, idx):
    nw = 32
    b_per_w = PAD_N // nw
    ch = 48
    n_ch = b_per_w // ch
    mesh = plsc.VectorSubcoreMesh(core_axis_name="c", subcore_axis_name="s")

    @functools.partial(
        pl.kernel, mesh=mesh,
        out_type=jax.ShapeDtypeStruct((PAD_N, HIDDEN), jnp.float32),
        scratch_types=[
            pltpu.VMEM((b_per_w,), jnp.int32),
            pltpu.VMEM((ch, HIDDEN), jnp.float32),
            pltpu.VMEM((ch, HIDDEN), jnp.float32),
            pltpu.SemaphoreType.DMA,
            pltpu.SemaphoreType.DMA,
        ],
    )
    def k(table_hbm, idx_hbm, out_hbm, idx_v, rows_a, rows_b, sem_a, sem_b):
        wid = jax.lax.axis_index("s") * 2 + jax.lax.axis_index("c")
        base = wid * b_per_w
        pltpu.sync_copy(idx_hbm.at[pl.ds(base, b_per_w)], idx_v)
        bufs = [(rows_a, sem_a), (rows_b, sem_b)]
        pend = [None, None]
        for c in range(n_ch):
            rv, sem = bufs[c % 2]
            pend[c % 2] = pltpu.async_copy(
                table_hbm.at[idx_v.at[pl.ds(c * ch, ch)]], rv, sem)
            pc = (c - 1) % 2
            if c >= 1:
                pend[pc].wait()
                pltpu.sync_copy(bufs[pc][0],
                                out_hbm.at[pl.ds(base + (c - 1) * ch, ch)])
        last = n_ch - 1
        pend[last % 2].wait()
        pltpu.sync_copy(bufs[last % 2][0],
                        out_hbm.at[pl.ds(base + last * ch, ch)])

    return k(table, idx)


def _route(probs):
    er = jnp.arange(E, dtype=jnp.int32)[None, :]
    m1 = jnp.max(probs, axis=-1)
    i1 = jnp.argmax(probs, axis=-1).astype(jnp.int32)
    masked = jnp.where(er == i1[:, None], -jnp.inf, probs)
    m2 = jnp.max(masked, axis=-1)
    i2 = jnp.argmax(masked, axis=-1).astype(jnp.int32)
    tw = jnp.stack([m1, m2], axis=-1)
    tw = tw / jnp.sum(tw, axis=-1, keepdims=True)
    flat_e = jnp.stack([i1, i2], axis=-1).reshape(-1)
    oh = (flat_e[:, None] == er).astype(jnp.int32)
    csum = jnp.cumsum(oh, axis=0)
    counts = csum[-1]
    rank = jnp.take_along_axis(csum, flat_e[:, None], axis=1)[:, 0] - 1
    padded = ((counts + MOE_T - 1) // MOE_T) * MOE_T
    pad_start = jnp.concatenate([jnp.zeros((1,), jnp.int32), jnp.cumsum(padded)[:-1]])
    pos = pad_start[flat_e] + rank
    row_token = jnp.zeros((PAD_N,), jnp.int32).at[pos].set(
        jnp.arange(N_ASSIGN, dtype=jnp.int32) // TOP_K)
    bounds = jnp.cumsum(padded)
    tile_expert = jnp.minimum(
        jnp.searchsorted(bounds, jnp.arange(NT, dtype=jnp.int32) * MOE_T, side='right'),
        E - 1).astype(jnp.int32)
    return tw, row_token, pos, tile_expert


def kernel(positions, hidden_states, ln1_w, qkv_w, o_w, ln_res_w, res_w13,
           res_w2, ln_post_w, gate_w, ws, w2s):
    q, kT, v = _qkv(positions, hidden_states, ln1_w, qkv_w)
    attn_o = _attention(q, kT, v)
    rm, hm, probs = _mid(attn_o, hidden_states, o_w, ln_res_w, res_w13,
                         res_w2, ln_post_w, gate_w)
    tw, row_token, pos, tile_expert = _route(probs)
    x_sorted = _sc_gather(hm, row_token)
    y = _moe_grouped(x_sorted, tile_expert, ws, w2s)
    pos2 = pos.reshape(SEQ, TOP_K)
    out = rm + tw[:, 0:1] * jnp.take(y, pos2[:, 0], axis=0) \
             + tw[:, 1:2] * jnp.take(y, pos2[:, 1], axis=0)
    return out


# speedup vs baseline: 1.0175x; 1.0175x over previous
import functools

import jax
import jax.numpy as jnp
from jax.experimental import pallas as pl
from jax.experimental.pallas import tpu as pltpu
from jax.experimental.pallas import tpu_sc as plsc

HIDDEN = 1024
N_HEADS = 16
N_KV = 4
HEAD_DIM = 64
FFN = 1024
E = 8
TOP_K = 2
SEQ = 2048
EPS = 1e-5
THETA = 10000.0

BQ = 512
BKV = 512
BS = 512
MOE_T = 128
N_ASSIGN = SEQ * TOP_K
NT = N_ASSIGN // MOE_T + E
PAD_N = NT * MOE_T


def _bdot(a, b):
    return jax.lax.dot_general(
        a.astype(jnp.bfloat16), b.astype(jnp.bfloat16),
        (((1,), (0,)), ((), ())), preferred_element_type=jnp.float32)


def _bdot_t(a, b):
    return jax.lax.dot_general(
        a.astype(jnp.bfloat16), b.astype(jnp.bfloat16),
        (((1,), (1,)), ((), ())), preferred_element_type=jnp.float32)


def _rms(x):
    return x * jax.lax.rsqrt(jnp.mean(x * x, axis=-1, keepdims=True) + EPS)


def _qkv_kernel(pos_ref, h_ref, ln1_ref, qkvw_ref, q_ref, k_ref, v_ref):
    h = h_ref[...]
    hn = _rms(h) * ln1_ref[0]
    qkv = _bdot_t(hn, qkvw_ref[...])
    half = HEAD_DIM // 2
    inv = THETA ** (-(jax.lax.iota(jnp.int32, half).astype(jnp.float32) * 2.0 / HEAD_DIM))
    pos = pos_ref[0].astype(jnp.float32)
    freqs = pos[:, None] * inv[None, :]
    cos = jnp.cos(freqs)
    sin = jnp.sin(freqs)
    scale = HEAD_DIM ** -0.5
    for hh in range(N_HEADS):
        b = hh * HEAD_DIM
        x1 = qkv[:, b:b + half]
        x2 = qkv[:, b + half:b + HEAD_DIM]
        q_ref[:, b:b + half] = ((x1 * cos - x2 * sin) * scale).astype(jnp.bfloat16)
        q_ref[:, b + half:b + HEAD_DIM] = ((x2 * cos + x1 * sin) * scale).astype(jnp.bfloat16)
    for hh in range(N_KV):
        b = N_HEADS * HEAD_DIM + hh * HEAD_DIM
        x1 = qkv[:, b:b + half]
        x2 = qkv[:, b + half:b + HEAD_DIM]
        k_ref[hh, :half, :] = (x1 * cos - x2 * sin).T.astype(jnp.bfloat16)
        k_ref[hh, half:, :] = (x2 * cos + x1 * sin).T.astype(jnp.bfloat16)
        vb = (N_HEADS + N_KV) * HEAD_DIM + hh * HEAD_DIM
        v_ref[hh, :, :HEAD_DIM] = qkv[:, vb:vb + HEAD_DIM].astype(jnp.bfloat16)
        v_ref[hh, :, HEAD_DIM:] = jnp.concatenate(
            [jnp.ones((BS, 1), jnp.bfloat16),
             jnp.zeros((BS, 127 - HEAD_DIM), jnp.bfloat16)], axis=1)


def _qkv(positions, hidden_states, ln1_w, qkv_w):
    nb = SEQ // BS
    return pl.pallas_call(
        _qkv_kernel,
        grid=(nb,),
        in_specs=[
            pl.BlockSpec((1, BS), lambda i: (0, i)),
            pl.BlockSpec((BS, HIDDEN), lambda i: (i, 0)),
            pl.BlockSpec((1, HIDDEN), lambda i: (0, 0)),
            pl.BlockSpec(((N_HEADS + 2 * N_KV) * HEAD_DIM, HIDDEN), lambda i: (0, 0)),
        ],
        out_specs=(
            pl.BlockSpec((BS, HIDDEN), lambda i: (i, 0)),
            pl.BlockSpec((N_KV, HEAD_DIM, BS), lambda i: (0, 0, i)),
            pl.BlockSpec((N_KV, BS, 128), lambda i: (0, i, 0)),
        ),
        out_shape=(
            jax.ShapeDtypeStruct((SEQ, HIDDEN), jnp.bfloat16),
            jax.ShapeDtypeStruct((N_KV, HEAD_DIM, SEQ), jnp.bfloat16),
            jax.ShapeDtypeStruct((N_KV, SEQ, 128), jnp.bfloat16),
        ),
    )(positions.reshape(1, SEQ), hidden_states, ln1_w.reshape(1, HIDDEN), qkv_w)


def _attn_kernel(q_ref, kT_ref, v_ref, o_ref, acc_ref, m_ref):
    qb = pl.program_id(0)
    j = pl.program_id(1)

    @pl.when(j == 0)
    def _init():
        m_ref[...] = jnp.full_like(m_ref, -1e30)
        acc_ref[...] = jnp.zeros_like(acc_ref)

    def _heads(masked):
        if masked:
            rows = qb * BQ + jax.lax.broadcasted_iota(jnp.int32, (BQ, BKV), 0)
            cols = j * BKV + jax.lax.broadcasted_iota(jnp.int32, (BQ, BKV), 1)
            bias = jnp.where(cols <= rows, jnp.float32(0), jnp.float32(-1e30))
        for hh in range(N_HEADS):
            g = hh // (N_HEADS // N_KV)
            b = hh * HEAD_DIM
            q = q_ref[:, b:b + HEAD_DIM]
            s = _bdot(q, kT_ref[g])
            if masked:
                s = s + bias
            m_prev = m_ref[hh]
            m_cur = jnp.max(s, axis=-1, keepdims=True)
            m_new = jnp.maximum(m_prev, jnp.broadcast_to(m_cur, m_prev.shape))
            alpha = jnp.exp(m_prev[:, :1] - m_new[:, :1])
            p = jnp.exp(s - m_new[:, :1])
            acc_ref[hh] = acc_ref[hh] * alpha + _bdot(p, v_ref[g])
            m_ref[hh] = m_new

    @pl.when(j < qb)
    def _body():
        _heads(False)

    @pl.when(j == qb)
    def _diag():
        _heads(True)

    @pl.when(j == pl.num_programs(1) - 1)
    def _fin():
        for hh in range(N_HEADS):
            b = hh * HEAD_DIM
            a = acc_ref[hh]
            o_ref[:, b:b + HEAD_DIM] = (a[:, :HEAD_DIM] / a[:, HEAD_DIM:HEAD_DIM + 1]
                                        ).astype(jnp.bfloat16)


def _attention(q, kT, v):
    nq = SEQ // BQ
    nkv = SEQ // BKV
    return pl.pallas_call(
        _attn_kernel,
        grid=(nq, nkv),
        in_specs=[
            pl.BlockSpec((BQ, HIDDEN), lambda i, j: (i, 0)),
            pl.BlockSpec((N_KV, HEAD_DIM, BKV), lambda i, j: (0, 0, jnp.minimum(j, i))),
            pl.BlockSpec((N_KV, BKV, 128), lambda i, j: (0, jnp.minimum(j, i), 0)),
        ],
        out_specs=pl.BlockSpec((BQ, HIDDEN), lambda i, j: (i, 0)),
        out_shape=jax.ShapeDtypeStruct((SEQ, HIDDEN), jnp.bfloat16),
        scratch_shapes=[
            pltpu.VMEM((N_HEADS, BQ, 128), jnp.float32),
            pltpu.VMEM((N_HEADS, BQ, 128), jnp.float32),
        ],
    )(q, kT, v)


def _mid_kernel(ao_ref, h0_ref, ow_ref, lnr_ref, w13_ref, w2_ref, lnp_ref,
                gw_ref, rm_ref, hm_ref, probs_ref):
    ao = ao_ref[...]
    ra = h0_ref[...] + _bdot_t(ao, ow_ref[...])
    hr = _rms(ra) * lnr_ref[0]
    g13 = _bdot_t(hr, w13_ref[...])
    g = g13[:, :HIDDEN]
    act = (g / (1.0 + jnp.exp(-g))) * g13[:, HIDDEN:]
    rm_ref[...] = ra + _bdot_t(act, w2_ref[...])
    hm = _rms(ra) * lnp_ref[0]
    hm_ref[...] = hm
    logits = jax.lax.dot_general(hm, gw_ref[...], (((1,), (1,)), ((), ())),
                                 preferred_element_type=jnp.float32)
    mx = jnp.max(logits, axis=-1, keepdims=True)
    ex = jnp.exp(logits - mx)
    probs_ref[...] = ex / jnp.sum(ex, axis=-1, keepdims=True)


def _mid(attn_o, hidden_states, o_w, ln_res_w, res_w13, res_w2, ln_post_w, gate_w):
    nb = SEQ // BS
    return pl.pallas_call(
        _mid_kernel,
        grid=(nb,),
        in_specs=[
            pl.BlockSpec((BS, HIDDEN), lambda i: (i, 0)),
            pl.BlockSpec((BS, HIDDEN), lambda i: (i, 0)),
            pl.BlockSpec((HIDDEN, N_HEADS * HEAD_DIM), lambda i: (0, 0)),
            pl.BlockSpec((1, HIDDEN), lambda i: (0, 0)),
            pl.BlockSpec((2 * HIDDEN, HIDDEN), lambda i: (0, 0)),
            pl.BlockSpec((HIDDEN, HIDDEN), lambda i: (0, 0)),
            pl.BlockSpec((1, HIDDEN), lambda i: (0, 0)),
            pl.BlockSpec((E, HIDDEN), lambda i: (0, 0)),
        ],
        out_specs=(
            pl.BlockSpec((BS, HIDDEN), lambda i: (i, 0)),
            pl.BlockSpec((BS, HIDDEN), lambda i: (i, 0)),
            pl.BlockSpec((BS, E), lambda i: (i, 0)),
        ),
        out_shape=(
            jax.ShapeDtypeStruct((SEQ, HIDDEN), jnp.float32),
            jax.ShapeDtypeStruct((SEQ, HIDDEN), jnp.float32),
            jax.ShapeDtypeStruct((SEQ, E), jnp.float32),
        ),
    )(attn_o, hidden_states, o_w, ln_res_w.reshape(1, HIDDEN), res_w13,
      res_w2, ln_post_w.reshape(1, HIDDEN), gate_w)


def _moe_kernel(te_ref, x_ref, ws_ref, w2s_ref, y_ref):
    x = x_ref[...]
    g13 = _bdot_t(x, ws_ref[0])
    g = g13[:, :FFN]
    act = (g / (1.0 + jnp.exp(-g))) * g13[:, FFN:]
    y_ref[...] = _bdot_t(act, w2s_ref[0])


def _moe_grouped(x_sorted, tile_expert, ws, w2s):
    grid_spec = pltpu.PrefetchScalarGridSpec(
        num_scalar_prefetch=1,
        grid=(NT,),
        in_specs=[
            pl.BlockSpec((MOE_T, HIDDEN), lambda t, te: (t, 0)),
            pl.BlockSpec((1, 2 * FFN, HIDDEN), lambda t, te: (te[t], 0, 0)),
            pl.BlockSpec((1, HIDDEN, FFN), lambda t, te: (te[t], 0, 0)),
        ],
        out_specs=pl.BlockSpec((MOE_T, HIDDEN), lambda t, te: (t, 0)),
    )
    return pl.pallas_call(
        _moe_kernel,
        grid_spec=grid_spec,
        out_shape=jax.ShapeDtypeStruct((PAD_N, HIDDEN), jnp.float32),
    )(tile_expert, x_sorted, ws, w2s)


def _sc_gather(table, idx):
    nw = 32
    b_per_w = PAD_N // nw
    ch = 48
    n_ch = b_per_w // ch
    mesh = plsc.VectorSubcoreMesh(core_axis_name="c", subcore_axis_name="s")

    @functools.partial(
        pl.kernel, mesh=mesh,
        out_type=jax.ShapeDtypeStruct((PAD_N, HIDDEN), jnp.float32),
        scratch_types=[
            pltpu.VMEM((b_per_w,), jnp.int32),
            pltpu.VMEM((ch, HIDDEN), jnp.float32),
            pltpu.VMEM((ch, HIDDEN), jnp.float32),
            pltpu.SemaphoreType.DMA,
            pltpu.SemaphoreType.DMA,
        ],
    )
    def k(table_hbm, idx_hbm, out_hbm, idx_v, rows_a, rows_b, sem_a, sem_b):
        wid = jax.lax.axis_index("s") * 2 + jax.lax.axis_index("c")
        base = wid * b_per_w
        pltpu.sync_copy(idx_hbm.at[pl.ds(base, b_per_w)], idx_v)
        bufs = [(rows_a, sem_a), (rows_b, sem_b)]
        pend = [None, None]
        for c in range(n_ch):
            rv, sem = bufs[c % 2]
            pend[c % 2] = pltpu.async_copy(
                table_hbm.at[idx_v.at[pl.ds(c * ch, ch)]], rv, sem)
            pc = (c - 1) % 2
            if c >= 1:
                pend[pc].wait()
                pltpu.sync_copy(bufs[pc][0],
                                out_hbm.at[pl.ds(base + (c - 1) * ch, ch)])
        last = n_ch - 1
        pend[last % 2].wait()
        pltpu.sync_copy(bufs[last % 2][0],
                        out_hbm.at[pl.ds(base + last * ch, ch)])

    return k(table, idx)


def _route(probs):
    er = jnp.arange(E, dtype=jnp.int32)[None, :]
    m1 = jnp.max(probs, axis=-1)
    i1 = jnp.argmax(probs, axis=-1).astype(jnp.int32)
    masked = jnp.where(er == i1[:, None], -jnp.inf, probs)
    m2 = jnp.max(masked, axis=-1)
    i2 = jnp.argmax(masked, axis=-1).astype(jnp.int32)
    tw = jnp.stack([m1, m2], axis=-1)
    tw = tw / jnp.sum(tw, axis=-1, keepdims=True)
    flat_e = jnp.stack([i1, i2], axis=-1).reshape(-1)
    oh = (flat_e[:, None] == er).astype(jnp.int32)
    csum = jnp.cumsum(oh, axis=0)
    counts = csum[-1]
    rank = jnp.take_along_axis(csum, flat_e[:, None], axis=1)[:, 0] - 1
    padded = ((counts + MOE_T - 1) // MOE_T) * MOE_T
    pad_start = jnp.concatenate([jnp.zeros((1,), jnp.int32), jnp.cumsum(padded)[:-1]])
    pos = pad_start[flat_e] + rank
    row_token = jnp.zeros((PAD_N,), jnp.int32).at[pos].set(
        jnp.arange(N_ASSIGN, dtype=jnp.int32) // TOP_K)
    bounds = jnp.cumsum(padded)
    tile_expert = jnp.minimum(
        jnp.searchsorted(bounds, jnp.arange(NT, dtype=jnp.int32) * MOE_T, side='right'),
        E - 1).astype(jnp.int32)
    return tw, row_token, pos, tile_expert


def kernel(positions, hidden_states, ln1_w, qkv_w, o_w, ln_res_w, res_w13,
           res_w2, ln_post_w, gate_w, ws, w2s):
    q, kT, v = _qkv(positions, hidden_states, ln1_w, qkv_w)
    attn_o = _attention(q, kT, v)
    rm, hm, probs = _mid(attn_o, hidden_states, o_w, ln_res_w, res_w13,
                         res_w2, ln_post_w, gate_w)
    tw, row_token, pos, tile_expert = _route(probs)
    x_sorted = jnp.take(hm, row_token, axis=0)
    y = _moe_grouped(x_sorted, tile_expert, ws, w2s)
    pos2 = pos.reshape(SEQ, TOP_K)
    out = rm + tw[:, 0:1] * jnp.take(y, pos2[:, 0], axis=0) \
             + tw[:, 1:2] * jnp.take(y, pos2[:, 1], axis=0)
    return out


# speedup vs baseline: 1.1497x; 1.1299x over previous
import jax
import jax.numpy as jnp
from jax.experimental import pallas as pl
from jax.experimental.pallas import tpu as pltpu

HIDDEN = 1024
N_HEADS = 16
N_KV = 4
HEAD_DIM = 64
FFN = 1024
E = 8
TOP_K = 2
SEQ = 2048
EPS = 1e-5
THETA = 10000.0

BQ = 512
BKV = 512
BS = 512
MOE_T = 256
N_ASSIGN = SEQ * TOP_K
NT = N_ASSIGN // MOE_T + E
PAD_N = NT * MOE_T


def _bdot(a, b):
    return jax.lax.dot_general(
        a.astype(jnp.bfloat16), b.astype(jnp.bfloat16),
        (((1,), (0,)), ((), ())), preferred_element_type=jnp.float32)


def _bdot_t(a, b):
    return jax.lax.dot_general(
        a.astype(jnp.bfloat16), b.astype(jnp.bfloat16),
        (((1,), (1,)), ((), ())), preferred_element_type=jnp.float32)


def _rms(x):
    return x * jax.lax.rsqrt(jnp.mean(x * x, axis=-1, keepdims=True) + EPS)


def _qkv_kernel(pos_ref, h_ref, ln1_ref, qkvw_ref, q_ref, k_ref, v_ref):
    h = h_ref[...]
    hn = _rms(h) * ln1_ref[0]
    qkv = _bdot_t(hn, qkvw_ref[...])
    half = HEAD_DIM // 2
    inv = THETA ** (-(jax.lax.iota(jnp.int32, half).astype(jnp.float32) * 2.0 / HEAD_DIM))
    pos = pos_ref[0].astype(jnp.float32)
    freqs = pos[:, None] * inv[None, :]
    cos = jnp.cos(freqs)
    sin = jnp.sin(freqs)
    scale = HEAD_DIM ** -0.5
    for hh in range(N_HEADS):
        b = hh * HEAD_DIM
        x1 = qkv[:, b:b + half]
        x2 = qkv[:, b + half:b + HEAD_DIM]
        q_ref[:, b:b + half] = ((x1 * cos - x2 * sin) * scale).astype(jnp.bfloat16)
        q_ref[:, b + half:b + HEAD_DIM] = ((x2 * cos + x1 * sin) * scale).astype(jnp.bfloat16)
    for hh in range(N_KV):
        b = N_HEADS * HEAD_DIM + hh * HEAD_DIM
        x1 = qkv[:, b:b + half]
        x2 = qkv[:, b + half:b + HEAD_DIM]
        k_ref[hh, :half, :] = (x1 * cos - x2 * sin).T.astype(jnp.bfloat16)
        k_ref[hh, half:, :] = (x2 * cos + x1 * sin).T.astype(jnp.bfloat16)
        vb = (N_HEADS + N_KV) * HEAD_DIM + hh * HEAD_DIM
        v_ref[hh, :, :HEAD_DIM] = qkv[:, vb:vb + HEAD_DIM].astype(jnp.bfloat16)
        v_ref[hh, :, HEAD_DIM:] = jnp.concatenate(
            [jnp.ones((BS, 1), jnp.bfloat16),
             jnp.zeros((BS, 127 - HEAD_DIM), jnp.bfloat16)], axis=1)


def _qkv(positions, hidden_states, ln1_w, qkv_w):
    nb = SEQ // BS
    return pl.pallas_call(
        _qkv_kernel,
        grid=(nb,),
        in_specs=[
            pl.BlockSpec((1, BS), lambda i: (0, i)),
            pl.BlockSpec((BS, HIDDEN), lambda i: (i, 0)),
            pl.BlockSpec((1, HIDDEN), lambda i: (0, 0)),
            pl.BlockSpec(((N_HEADS + 2 * N_KV) * HEAD_DIM, HIDDEN), lambda i: (0, 0)),
        ],
        out_specs=(
            pl.BlockSpec((BS, HIDDEN), lambda i: (i, 0)),
            pl.BlockSpec((N_KV, HEAD_DIM, BS), lambda i: (0, 0, i)),
            pl.BlockSpec((N_KV, BS, 128), lambda i: (0, i, 0)),
        ),
        out_shape=(
            jax.ShapeDtypeStruct((SEQ, HIDDEN), jnp.bfloat16),
            jax.ShapeDtypeStruct((N_KV, HEAD_DIM, SEQ), jnp.bfloat16),
            jax.ShapeDtypeStruct((N_KV, SEQ, 128), jnp.bfloat16),
        ),
    )(positions.reshape(1, SEQ), hidden_states, ln1_w.reshape(1, HIDDEN), qkv_w)


def _attn_kernel(q_ref, kT_ref, v_ref, o_ref, acc_ref, m_ref):
    qb = pl.program_id(0)
    j = pl.program_id(1)

    @pl.when(j == 0)
    def _init():
        m_ref[...] = jnp.full_like(m_ref, -1e30)
        acc_ref[...] = jnp.zeros_like(acc_ref)

    def _heads(masked):
        if masked:
            rows = qb * BQ + jax.lax.broadcasted_iota(jnp.int32, (BQ, BKV), 0)
            cols = j * BKV + jax.lax.broadcasted_iota(jnp.int32, (BQ, BKV), 1)
            bias = jnp.where(cols <= rows, jnp.float32(0), jnp.float32(-1e30))
        for hh in range(N_HEADS):
            g = hh // (N_HEADS // N_KV)
            b = hh * HEAD_DIM
            q = q_ref[:, b:b + HEAD_DIM]
            s = _bdot(q, kT_ref[g])
            if masked:
                s = s + bias
            m_prev = m_ref[hh]
            m_cur = jnp.max(s, axis=-1, keepdims=True)
            m_new = jnp.maximum(m_prev, jnp.broadcast_to(m_cur, m_prev.shape))
            alpha = jnp.exp(m_prev[:, :1] - m_new[:, :1])
            p = jnp.exp(s - m_new[:, :1])
            acc_ref[hh] = acc_ref[hh] * alpha + _bdot(p, v_ref[g])
            m_ref[hh] = m_new

    @pl.when(j < qb)
    def _body():
        _heads(False)

    @pl.when(j == qb)
    def _diag():
        _heads(True)

    @pl.when(j == pl.num_programs(1) - 1)
    def _fin():
        for hh in range(N_HEADS):
            b = hh * HEAD_DIM
            a = acc_ref[hh]
            o_ref[:, b:b + HEAD_DIM] = (a[:, :HEAD_DIM] / a[:, HEAD_DIM:HEAD_DIM + 1]
                                        ).astype(jnp.bfloat16)


def _attention(q, kT, v):
    nq = SEQ // BQ
    nkv = SEQ // BKV
    return pl.pallas_call(
        _attn_kernel,
        grid=(nq, nkv),
        in_specs=[
            pl.BlockSpec((BQ, HIDDEN), lambda i, j: (i, 0)),
            pl.BlockSpec((N_KV, HEAD_DIM, BKV), lambda i, j: (0, 0, jnp.minimum(j, i))),
            pl.BlockSpec((N_KV, BKV, 128), lambda i, j: (0, jnp.minimum(j, i), 0)),
        ],
        out_specs=pl.BlockSpec((BQ, HIDDEN), lambda i, j: (i, 0)),
        out_shape=jax.ShapeDtypeStruct((SEQ, HIDDEN), jnp.bfloat16),
        scratch_shapes=[
            pltpu.VMEM((N_HEADS, BQ, 128), jnp.float32),
            pltpu.VMEM((N_HEADS, BQ, 128), jnp.float32),
        ],
    )(q, kT, v)


def _mid_kernel(ao_ref, h0_ref, ow_ref, lnr_ref, w13_ref, w2_ref, lnp_ref,
                gw_ref, rm_ref, hm_ref, probs_ref):
    ao = ao_ref[...]
    ra = h0_ref[...] + _bdot_t(ao, ow_ref[...])
    hr = _rms(ra) * lnr_ref[0]
    g13 = _bdot_t(hr, w13_ref[...])
    g = g13[:, :HIDDEN]
    act = (g / (1.0 + jnp.exp(-g))) * g13[:, HIDDEN:]
    rm_ref[...] = ra + _bdot_t(act, w2_ref[...])
    hm = _rms(ra) * lnp_ref[0]
    hm_ref[...] = hm
    logits = jax.lax.dot_general(hm, gw_ref[...], (((1,), (1,)), ((), ())),
                                 preferred_element_type=jnp.float32)
    mx = jnp.max(logits, axis=-1, keepdims=True)
    ex = jnp.exp(logits - mx)
    probs_ref[...] = ex / jnp.sum(ex, axis=-1, keepdims=True)


def _mid(attn_o, hidden_states, o_w, ln_res_w, res_w13, res_w2, ln_post_w, gate_w):
    nb = SEQ // BS
    return pl.pallas_call(
        _mid_kernel,
        grid=(nb,),
        in_specs=[
            pl.BlockSpec((BS, HIDDEN), lambda i: (i, 0)),
            pl.BlockSpec((BS, HIDDEN), lambda i: (i, 0)),
            pl.BlockSpec((HIDDEN, N_HEADS * HEAD_DIM), lambda i: (0, 0)),
            pl.BlockSpec((1, HIDDEN), lambda i: (0, 0)),
            pl.BlockSpec((2 * HIDDEN, HIDDEN), lambda i: (0, 0)),
            pl.BlockSpec((HIDDEN, HIDDEN), lambda i: (0, 0)),
            pl.BlockSpec((1, HIDDEN), lambda i: (0, 0)),
            pl.BlockSpec((E, HIDDEN), lambda i: (0, 0)),
        ],
        out_specs=(
            pl.BlockSpec((BS, HIDDEN), lambda i: (i, 0)),
            pl.BlockSpec((BS, HIDDEN), lambda i: (i, 0)),
            pl.BlockSpec((BS, E), lambda i: (i, 0)),
        ),
        out_shape=(
            jax.ShapeDtypeStruct((SEQ, HIDDEN), jnp.float32),
            jax.ShapeDtypeStruct((SEQ, HIDDEN), jnp.float32),
            jax.ShapeDtypeStruct((SEQ, E), jnp.float32),
        ),
    )(attn_o, hidden_states, o_w, ln_res_w.reshape(1, HIDDEN), res_w13,
      res_w2, ln_post_w.reshape(1, HIDDEN), gate_w)


def _moe_kernel(te_ref, x_ref, ws_ref, w2s_ref, y_ref):
    x = x_ref[...]
    g13 = _bdot_t(x, ws_ref[0])
    g = g13[:, :FFN]
    act = (g / (1.0 + jnp.exp(-g))) * g13[:, FFN:]
    y_ref[...] = _bdot_t(act, w2s_ref[0])


def _moe_grouped(x_sorted, tile_expert, ws, w2s):
    grid_spec = pltpu.PrefetchScalarGridSpec(
        num_scalar_prefetch=1,
        grid=(NT,),
        in_specs=[
            pl.BlockSpec((MOE_T, HIDDEN), lambda t, te: (t, 0)),
            pl.BlockSpec((1, 2 * FFN, HIDDEN), lambda t, te: (te[t], 0, 0)),
            pl.BlockSpec((1, HIDDEN, FFN), lambda t, te: (te[t], 0, 0)),
        ],
        out_specs=pl.BlockSpec((MOE_T, HIDDEN), lambda t, te: (t, 0)),
    )
    return pl.pallas_call(
        _moe_kernel,
        grid_spec=grid_spec,
        out_shape=jax.ShapeDtypeStruct((PAD_N, HIDDEN), jnp.float32),
    )(tile_expert, x_sorted, ws, w2s)


def _route(probs):
    er = jnp.arange(E, dtype=jnp.int32)[None, :]
    m1 = jnp.max(probs, axis=-1)
    i1 = jnp.argmax(probs, axis=-1).astype(jnp.int32)
    masked = jnp.where(er == i1[:, None], -jnp.inf, probs)
    m2 = jnp.max(masked, axis=-1)
    i2 = jnp.argmax(masked, axis=-1).astype(jnp.int32)
    tw = jnp.stack([m1, m2], axis=-1)
    tw = tw / jnp.sum(tw, axis=-1, keepdims=True)
    flat_e = jnp.stack([i1, i2], axis=-1).reshape(-1)
    oh = (flat_e[:, None] == er).astype(jnp.int32)
    csum = jnp.cumsum(oh, axis=0)
    counts = csum[-1]
    rank = jnp.take_along_axis(csum, flat_e[:, None], axis=1)[:, 0] - 1
    padded = ((counts + MOE_T - 1) // MOE_T) * MOE_T
    pad_start = jnp.concatenate([jnp.zeros((1,), jnp.int32), jnp.cumsum(padded)[:-1]])
    pos = pad_start[flat_e] + rank
    row_token = jnp.zeros((PAD_N,), jnp.int32).at[pos].set(
        jnp.arange(N_ASSIGN, dtype=jnp.int32) // TOP_K)
    bounds = jnp.cumsum(padded)
    tile_expert = jnp.minimum(
        jnp.searchsorted(bounds, jnp.arange(NT, dtype=jnp.int32) * MOE_T, side='right'),
        E - 1).astype(jnp.int32)
    return tw, row_token, pos, tile_expert


def kernel(positions, hidden_states, ln1_w, qkv_w, o_w, ln_res_w, res_w13,
           res_w2, ln_post_w, gate_w, ws, w2s):
    q, kT, v = _qkv(positions, hidden_states, ln1_w, qkv_w)
    attn_o = _attention(q, kT, v)
    rm, hm, probs = _mid(attn_o, hidden_states, o_w, ln_res_w, res_w13,
                         res_w2, ln_post_w, gate_w)
    tw, row_token, pos, tile_expert = _route(probs)
    x_sorted = jnp.take(hm, row_token, axis=0)
    y = _moe_grouped(x_sorted, tile_expert, ws, w2s)
    pos2 = pos.reshape(SEQ, TOP_K)
    out = rm + tw[:, 0:1] * jnp.take(y, pos2[:, 0], axis=0) \
             + tw[:, 1:2] * jnp.take(y, pos2[:, 1], axis=0)
    return out
